# baseline (speedup 1.0000x reference)
import functools

import jax
import jax.numpy as jnp
from jax.experimental import pallas as pl
from jax.experimental.pallas import tpu as pltpu

_N = 16 * 576 * 96
_ROWS = _N // 2
_BR = 1024


def _body(x_ref, c_ref, out_ref, q_ref):
    x2 = x_ref[...]
    c = c_ref[...]
    lane = jax.lax.broadcasted_iota(jnp.int32, (x2.shape[0], 128), 1)
    left = lane < 64
    xa = jnp.where(left, x2[:, 0:1], x2[:, 1:2])
    e = jnp.exp(-jnp.abs(xa - c))
    w = e * c
    s0 = jnp.sum(e[:, :64], axis=1, keepdims=True)
    s1 = jnp.sum(e[:, 64:], axis=1, keepdims=True)
    n0 = jnp.sum(w[:, :64], axis=1, keepdims=True)
    n1 = jnp.sum(w[:, 64:], axis=1, keepdims=True)
    r0 = 1.0 / s0
    r1 = 1.0 / s1
    out_ref[...] = e * jnp.where(left, r0, r1)
    q_ref[...] = jnp.concatenate([n0 * r0, n1 * r1], axis=1)


@jax.jit
def kernel(x, centers):
    x2 = x.reshape(_ROWS, 2)
    ct = jnp.concatenate([centers, centers]).reshape(1, 128)
    grid = _ROWS // _BR
    out2, q2 = pl.pallas_call(
        _body,
        grid=(grid,),
        in_specs=[
            pl.BlockSpec((_BR, 2), lambda i: (i, 0)),
            pl.BlockSpec((1, 128), lambda i: (0, 0)),
        ],
        out_specs=[
            pl.BlockSpec((_BR, 128), lambda i: (i, 0)),
            pl.BlockSpec((_BR, 2), lambda i: (i, 0)),
        ],
        out_shape=[
            jax.ShapeDtypeStruct((_ROWS, 128), jnp.float32),
            jax.ShapeDtypeStruct((_ROWS, 2), jnp.float32),
        ],
    )(x2, ct)
    assign = out2.reshape(*x.shape, 64)
    quant = q2.reshape(x.shape)
    return quant, assign
